# baseline (speedup 1.0000x reference)
import functools

import jax
import jax.numpy as jnp
from jax import lax
from jax.experimental import pallas as pl
from jax.experimental.pallas import tpu as pltpu
from jax.experimental.pallas import tpu_sc as plsc

D = 128
DI = 64
L = 16
NC, NS = 2, 16
NW = NC * NS
E = 320000
ZN = 10000
EPW = E // NW
C = 320
NFULL = EPW // C
TAIL = EPW - NFULL * C
ZR = ZN // NS
PR = 25
NPCH = ZR // PR


def _decoder_body(z_hbm, ei_hbm, out_hbm, zi_hbm,
                  zin, zout,
                  sidx_all, didx_all,
                  srow0, drow0, srow1, drow1,
                  out_v, sem_i,
                  sem_s0, sem_d0, sem_s1, sem_d1):
    core = lax.axis_index("c")
    sub = lax.axis_index("s")
    wid = sub * NC + core
    base = wid * EPW
    lane = lax.iota(jnp.int32, L)

    cp_si = pltpu.async_copy(ei_hbm.at[0, pl.ds(base, EPW)], sidx_all, sem_i)
    cp_di = pltpu.async_copy(ei_hbm.at[1, pl.ds(base, EPW)], didx_all, sem_i)

    zrow_base = sub * ZR

    def pack_chunk(pc, carry):
        rb = zrow_base + pc * PR
        pltpu.sync_copy(z_hbm.at[pl.ds(rb, PR)], zin)

        def pack_row(r, carry2):
            for r2 in range(5):
                e = r * 5 + r2
                for j in range(DI // L):
                    a = zin[e, pl.ds(j * L, L)]
                    b = zin[e, pl.ds(D // 2 + j * L, L)]
                    p = plsc.pack(a, b, format=plsc.PackFormat.INTERLEAVED)
                    zout[e, pl.ds(j * L, L)] = plsc.bitcast(p, jnp.int32)
            return carry2

        lax.fori_loop(0, PR // 5, pack_row, 0)
        pltpu.sync_copy(zout, zi_hbm.at[core, pl.ds(rb, PR)])
        return carry

    lax.fori_loop(0, NPCH, pack_chunk, 0)
    plsc.subcore_barrier()
    cp_si.wait()
    cp_di.wait()

    zt = zi_hbm.at[core]
    srow = (srow0, srow1)
    drow = (drow0, drow1)
    sem_s = (sem_s0, sem_s1)
    sem_d = (sem_d0, sem_d1)

    def issue(g, b):
        pltpu.async_copy(zt.at[sidx_all.at[pl.ds(g * C, C)]], srow[b], sem_s[b])
        pltpu.async_copy(zt.at[didx_all.at[pl.ds(g * C, C)]], drow[b], sem_d[b])

    def wait(g, b):
        pltpu.make_async_copy(
            zt.at[sidx_all.at[pl.ds(g * C, C)]], srow[b], sem_s[b]).wait()
        pltpu.make_async_copy(
            zt.at[didx_all.at[pl.ds(g * C, C)]], drow[b], sem_d[b]).wait()

    def compute(g, b):
        sr, dr = srow[b], drow[b]

        def group_body(t, carry2):
            vec = jnp.zeros((L,), jnp.float32)
            for k in range(L):
                e = t * L + k
                acc0 = jnp.zeros((L,), jnp.float32)
                acc1 = jnp.zeros((L,), jnp.float32)
                for j in range(0, DI // L, 2):
                    a0 = plsc.bitcast(sr[e, pl.ds(j * L, L)], jnp.bfloat16)
                    b0 = plsc.bitcast(dr[e, pl.ds(j * L, L)], jnp.bfloat16)
                    a1 = plsc.bitcast(sr[e, pl.ds((j + 1) * L, L)], jnp.bfloat16)
                    b1 = plsc.bitcast(dr[e, pl.ds((j + 1) * L, L)], jnp.bfloat16)
                    s = a0 * b0 + a1 * b1
                    p0, p1 = plsc.unpack(s, format=plsc.PackFormat.INTERLEAVED)
                    acc0 += p0
                    acc1 += p1
                vec = jnp.where(lane == k, jnp.sum(acc0 + acc1), vec)
            out_v[pl.ds(g * C + t * L, L)] = 1.0 / (1.0 + jnp.exp(-vec))
            return carry2

        lax.fori_loop(0, C // L, group_body, 0)

    def issue_tail(b):
        tb = NFULL * C
        pltpu.async_copy(
            zt.at[sidx_all.at[pl.ds(tb, TAIL)]],
            srow[b].at[pl.ds(0, TAIL)], sem_s[b])
        pltpu.async_copy(
            zt.at[didx_all.at[pl.ds(tb, TAIL)]],
            drow[b].at[pl.ds(0, TAIL)], sem_d[b])

    def wait_tail(b):
        tb = NFULL * C
        pltpu.make_async_copy(
            zt.at[sidx_all.at[pl.ds(tb, TAIL)]],
            srow[b].at[pl.ds(0, TAIL)], sem_s[b]).wait()
        pltpu.make_async_copy(
            zt.at[didx_all.at[pl.ds(tb, TAIL)]],
            drow[b].at[pl.ds(0, TAIL)], sem_d[b]).wait()

    issue(0, 0)

    def pair_body(i, carry):
        for b in range(2):
            g = 2 * i + b
            wait(g, b)
            issue(g + 1, 1 - b)
            compute(g, b)
        return carry

    NMAIN = ((NFULL - 2) // 2) * 2
    lax.fori_loop(0, NMAIN // 2, pair_body, 0)
    for g in range(NMAIN, NFULL):
        wait(g, g % 2)
        if g + 1 < NFULL:
            issue(g + 1, (g + 1) % 2)
        elif TAIL:
            issue_tail((g + 1) % 2)
        compute(g, g % 2)
    if TAIL:
        bt = NFULL % 2
        wait_tail(bt)
        sr, dr = srow[bt], drow[bt]
        for t in range(TAIL // L):
            vec = jnp.zeros((L,), jnp.float32)
            for k in range(L):
                e = t * L + k
                acc0 = jnp.zeros((L,), jnp.float32)
                acc1 = jnp.zeros((L,), jnp.float32)
                for j in range(0, DI // L, 2):
                    a0 = plsc.bitcast(sr[e, pl.ds(j * L, L)], jnp.bfloat16)
                    b0 = plsc.bitcast(dr[e, pl.ds(j * L, L)], jnp.bfloat16)
                    a1 = plsc.bitcast(sr[e, pl.ds((j + 1) * L, L)], jnp.bfloat16)
                    b1 = plsc.bitcast(dr[e, pl.ds((j + 1) * L, L)], jnp.bfloat16)
                    s = a0 * b0 + a1 * b1
                    p0, p1 = plsc.unpack(s, format=plsc.PackFormat.INTERLEAVED)
                    acc0 += p0
                    acc1 += p1
                vec = jnp.where(lane == k, jnp.sum(acc0 + acc1), vec)
            out_v[pl.ds(NFULL * C + t * L, L)] = 1.0 / (1.0 + jnp.exp(-vec))

    pltpu.sync_copy(out_v, out_hbm.at[pl.ds(base, EPW)])


_decoder = functools.partial(
    pl.kernel,
    out_type=(
        jax.ShapeDtypeStruct((E,), jnp.float32),
        jax.ShapeDtypeStruct((NC, ZN, DI), jnp.int32),
    ),
    mesh=plsc.VectorSubcoreMesh(core_axis_name="c", subcore_axis_name="s"),
    compiler_params=pltpu.CompilerParams(
        needs_layout_passes=False, use_tc_tiling_on_sc=False,
        disable_bounds_checks=True, disable_semaphore_checks=True),
    scratch_types=[
        pltpu.VMEM((PR, D), jnp.float32),
        pltpu.VMEM((PR, DI), jnp.int32),
        pltpu.VMEM((EPW,), jnp.int32),
        pltpu.VMEM((EPW,), jnp.int32),
        pltpu.VMEM((C, DI), jnp.int32),
        pltpu.VMEM((C, DI), jnp.int32),
        pltpu.VMEM((C, DI), jnp.int32),
        pltpu.VMEM((C, DI), jnp.int32),
        pltpu.VMEM((EPW,), jnp.float32),
        pltpu.SemaphoreType.DMA,
        pltpu.SemaphoreType.DMA,
        pltpu.SemaphoreType.DMA,
        pltpu.SemaphoreType.DMA,
        pltpu.SemaphoreType.DMA,
    ],
)(_decoder_body)


def kernel(z, edge_index):
    out, _ = _decoder(z, edge_index.astype(jnp.int32))
    return out


# speedup vs baseline: 1.1003x; 1.1003x over previous
import functools

import jax
import jax.numpy as jnp
from jax import lax
from jax.experimental import pallas as pl
from jax.experimental.pallas import tpu as pltpu
from jax.experimental.pallas import tpu_sc as plsc

D = 128
DI = 64
L = 16
NC, NS = 2, 16
NW = NC * NS
E = 320000
ZN = 10000
EPW = E // NW
C = 256
NFULL = EPW // C
TAIL = EPW - NFULL * C
ZR = ZN // NS
PR = 125
NPCH = ZR // PR


def _decoder_body(z_hbm, ei_hbm, out_hbm, zi_hbm,
                  zin, zout,
                  sidx_all, didx_all,
                  srow0, drow0, srow1, drow1,
                  out_v, sem_i,
                  sem_s0, sem_d0, sem_s1, sem_d1):
    core = lax.axis_index("c")
    sub = lax.axis_index("s")
    wid = sub * NC + core
    base = wid * EPW
    lane = lax.iota(jnp.int32, L)

    cp_si = pltpu.async_copy(ei_hbm.at[0, pl.ds(base, EPW)], sidx_all, sem_i)
    cp_di = pltpu.async_copy(ei_hbm.at[1, pl.ds(base, EPW)], didx_all, sem_i)

    zrow_base = sub * ZR

    def pack_chunk(pc, carry):
        rb = zrow_base + pc * PR
        pltpu.sync_copy(z_hbm.at[pl.ds(rb, PR)], zin)

        def pack_row(r, carry2):
            for r2 in range(5):
                e = r * 5 + r2
                for j in range(DI // L):
                    a = zin[e, pl.ds(j * L, L)]
                    b = zin[e, pl.ds(D // 2 + j * L, L)]
                    p = plsc.pack(a, b, format=plsc.PackFormat.INTERLEAVED)
                    zout[e, pl.ds(j * L, L)] = plsc.bitcast(p, jnp.int32)
            return carry2

        lax.fori_loop(0, PR // 5, pack_row, 0)
        pltpu.sync_copy(zout, zi_hbm.at[core, pl.ds(rb, PR)])
        return carry

    lax.fori_loop(0, NPCH, pack_chunk, 0)
    plsc.subcore_barrier()
    cp_si.wait()
    cp_di.wait()

    zt = zi_hbm.at[core]
    srow = (srow0, srow1)
    drow = (drow0, drow1)
    sem_s = (sem_s0, sem_s1)
    sem_d = (sem_d0, sem_d1)

    def issue(g, b):
        pltpu.async_copy(zt.at[sidx_all.at[pl.ds(g * C, C)]], srow[b], sem_s[b])
        pltpu.async_copy(zt.at[didx_all.at[pl.ds(g * C, C)]], drow[b], sem_d[b])

    def wait(g, b):
        pltpu.make_async_copy(
            zt.at[sidx_all.at[pl.ds(g * C, C)]], srow[b], sem_s[b]).wait()
        pltpu.make_async_copy(
            zt.at[didx_all.at[pl.ds(g * C, C)]], drow[b], sem_d[b]).wait()

    def compute(g, b):
        sr, dr = srow[b], drow[b]

        def group_body(t, carry2):
            vec = jnp.zeros((L,), jnp.float32)
            for k in range(L):
                e = t * L + k
                acc0 = jnp.zeros((L,), jnp.float32)
                acc1 = jnp.zeros((L,), jnp.float32)
                for j in range(0, DI // L, 2):
                    a0 = plsc.bitcast(sr[e, pl.ds(j * L, L)], jnp.bfloat16)
                    b0 = plsc.bitcast(dr[e, pl.ds(j * L, L)], jnp.bfloat16)
                    a1 = plsc.bitcast(sr[e, pl.ds((j + 1) * L, L)], jnp.bfloat16)
                    b1 = plsc.bitcast(dr[e, pl.ds((j + 1) * L, L)], jnp.bfloat16)
                    s = a0 * b0 + a1 * b1
                    p0, p1 = plsc.unpack(s, format=plsc.PackFormat.INTERLEAVED)
                    acc0 += p0
                    acc1 += p1
                vec = jnp.where(lane == k, jnp.sum(acc0 + acc1), vec)
            out_v[pl.ds(g * C + t * L, L)] = 1.0 / (1.0 + jnp.exp(-vec))
            return carry2

        lax.fori_loop(0, C // L, group_body, 0)

    def issue_tail(b):
        tb = NFULL * C
        pltpu.async_copy(
            zt.at[sidx_all.at[pl.ds(tb, TAIL)]],
            srow[b].at[pl.ds(0, TAIL)], sem_s[b])
        pltpu.async_copy(
            zt.at[didx_all.at[pl.ds(tb, TAIL)]],
            drow[b].at[pl.ds(0, TAIL)], sem_d[b])

    def wait_tail(b):
        tb = NFULL * C
        pltpu.make_async_copy(
            zt.at[sidx_all.at[pl.ds(tb, TAIL)]],
            srow[b].at[pl.ds(0, TAIL)], sem_s[b]).wait()
        pltpu.make_async_copy(
            zt.at[didx_all.at[pl.ds(tb, TAIL)]],
            drow[b].at[pl.ds(0, TAIL)], sem_d[b]).wait()

    issue(0, 0)

    def pair_body(i, carry):
        for b in range(2):
            g = 2 * i + b
            wait(g, b)
            issue(g + 1, 1 - b)
            compute(g, b)
        return carry

    NMAIN = ((NFULL - 2) // 2) * 2
    lax.fori_loop(0, NMAIN // 2, pair_body, 0)
    for g in range(NMAIN, NFULL):
        wait(g, g % 2)
        if g + 1 < NFULL:
            issue(g + 1, (g + 1) % 2)
        elif TAIL:
            issue_tail((g + 1) % 2)
        compute(g, g % 2)
    if TAIL:
        bt = NFULL % 2
        wait_tail(bt)
        sr, dr = srow[bt], drow[bt]
        for t in range(TAIL // L):
            vec = jnp.zeros((L,), jnp.float32)
            for k in range(L):
                e = t * L + k
                acc0 = jnp.zeros((L,), jnp.float32)
                acc1 = jnp.zeros((L,), jnp.float32)
                for j in range(0, DI // L, 2):
                    a0 = plsc.bitcast(sr[e, pl.ds(j * L, L)], jnp.bfloat16)
                    b0 = plsc.bitcast(dr[e, pl.ds(j * L, L)], jnp.bfloat16)
                    a1 = plsc.bitcast(sr[e, pl.ds((j + 1) * L, L)], jnp.bfloat16)
                    b1 = plsc.bitcast(dr[e, pl.ds((j + 1) * L, L)], jnp.bfloat16)
                    s = a0 * b0 + a1 * b1
                    p0, p1 = plsc.unpack(s, format=plsc.PackFormat.INTERLEAVED)
                    acc0 += p0
                    acc1 += p1
                vec = jnp.where(lane == k, jnp.sum(acc0 + acc1), vec)
            out_v[pl.ds(NFULL * C + t * L, L)] = 1.0 / (1.0 + jnp.exp(-vec))

    pltpu.sync_copy(out_v, out_hbm.at[pl.ds(base, EPW)])


_decoder = functools.partial(
    pl.kernel,
    out_type=(
        jax.ShapeDtypeStruct((E,), jnp.float32),
        jax.ShapeDtypeStruct((NC, ZN, DI), jnp.int32),
    ),
    mesh=plsc.VectorSubcoreMesh(core_axis_name="c", subcore_axis_name="s"),
    compiler_params=pltpu.CompilerParams(
        needs_layout_passes=False, use_tc_tiling_on_sc=False,
        disable_bounds_checks=True, disable_semaphore_checks=True),
    scratch_types=[
        pltpu.VMEM((PR, D), jnp.float32),
        pltpu.VMEM((PR, DI), jnp.int32),
        pltpu.VMEM((EPW,), jnp.int32),
        pltpu.VMEM((EPW,), jnp.int32),
        pltpu.VMEM((C, DI), jnp.int32),
        pltpu.VMEM((C, DI), jnp.int32),
        pltpu.VMEM((C, DI), jnp.int32),
        pltpu.VMEM((C, DI), jnp.int32),
        pltpu.VMEM((EPW,), jnp.float32),
        pltpu.SemaphoreType.DMA,
        pltpu.SemaphoreType.DMA,
        pltpu.SemaphoreType.DMA,
        pltpu.SemaphoreType.DMA,
        pltpu.SemaphoreType.DMA,
    ],
)(_decoder_body)


def kernel(z, edge_index):
    out, _ = _decoder(z, edge_index.astype(jnp.int32))
    return out
